# baseline (speedup 1.0000x reference)
import dataclasses

import jax
import jax.numpy as jnp
from jax import lax
from jax.experimental import pallas as pl
from jax.experimental.pallas import tpu as pltpu
from jax.experimental.pallas import tpu_sc as plsc

N = 10000
D = 128
E = 320000

NC = 2
NS = 16
NW = NC * NS
CHUNK = 128
LANES = 16

NCHT = ((-(-E // (NS * CHUNK)) + 15) // 16) * 16
EV = NCHT * CHUNK
E_PAD2 = EV * NS

N_PAD = ((N + NS * CHUNK) // (NS * CHUNK)) * NS * CHUNK
RPT = N_PAD // NS
TRASH = 16383

HN = N_PAD // NC
Z_ROWS = HN + CHUNK
LTRASH = HN
ZPT = Z_ROWS // NS
WPT = HN // NS


def _mesh():
    return plsc.VectorSubcoreMesh(core_axis_name="c", subcore_axis_name="s",
                                  num_cores=NC, num_subcores=NS)


HB = 128


def _cp():
    cp = pltpu.CompilerParams()
    if "needs_layout_passes" in pltpu.CompilerParams.__dataclass_fields__:
        cp = dataclasses.replace(cp, needs_layout_passes=False)
    return cp


def _deg_body(dst_hbm, deg_hbm, dstv, idxrow, histo, deg_sh):
    cid = lax.axis_index("c")
    sid = lax.axis_index("s")
    pltpu.sync_copy(dst_hbm.at[sid, pl.ds(cid * (EV // 2), EV // 2)], dstv)

    @pl.loop(0, HB)
    def _zr(i):
        @pl.loop(0, D, step=LANES)
        def _zc(c):
            histo[i, pl.ds(c, LANES)] = jnp.zeros((LANES,), jnp.float32)

    @pl.loop(0, D, step=LANES)
    def _io(c):
        idxrow[0, pl.ds(c, LANES)] = lax.iota(jnp.int32, LANES) + c

    pltpu.sync_copy(histo.at[pl.ds(sid * (HB // NS), HB // NS)],
                    deg_sh.at[pl.ds(sid * (HB // NS), HB // NS)])
    plsc.subcore_barrier()

    @pl.loop(0, EV // 2, step=LANES)
    def _hist(c):
        v = dstv[pl.ds(c, LANES)]
        plsc.addupdate_scatter(histo, [v >> 7, v & 127],
                               jnp.ones((LANES,), jnp.float32))

    pltpu.sync_copy(histo, deg_sh.at[idxrow.at[0]], add=True)
    plsc.subcore_barrier()
    pltpu.sync_copy(deg_sh.at[pl.ds(sid * (HB // NS), HB // NS)],
                    deg_hbm.at[cid, pl.ds(sid * (HB // NS), HB // NS)])


@jax.jit
def _deg_call(dst_slab):
    k = pl.kernel(
        _deg_body,
        out_type=jax.ShapeDtypeStruct((NC, HB, D), jnp.float32),
        mesh=_mesh(),
        compiler_params=_cp(),
        scratch_types=[
            pltpu.VMEM((EV // 2,), jnp.int32),
            pltpu.VMEM((1, D), jnp.int32),
            pltpu.VMEM((HB, D), jnp.float32),
            pltpu.VMEM_SHARED((HB, D), jnp.float32),
        ],
    )
    return k(dst_slab)


def _agg_body(xs_hbm, src_hbm, dst_hbm, z_hbm, srcv, dstv, rb0, rb1, z_sh,
              sem0, sem1, sem2, sem3):
    cid = lax.axis_index("c")
    sid = lax.axis_index("s")
    base = cid * HN
    pltpu.sync_copy(src_hbm.at[sid], srcv.at[pl.ds(0, EV)])
    pltpu.sync_copy(dst_hbm.at[sid], dstv.at[pl.ds(0, EV)])

    def _compact(i, off):
        v = dstv[pl.ds(i * LANES, LANES)]
        s = srcv[pl.ds(i * LANES, LANES)]
        lv = v - base
        ok = (lv >= 0) & (lv < HN)
        plsc.store_compressed(dstv.at[pl.ds(off, LANES)], lv, mask=ok)
        plsc.store_compressed(srcv.at[pl.ds(off, LANES)], s, mask=ok)
        return off + jnp.sum(ok.astype(jnp.int32))

    cnt = lax.fori_loop(0, EV // LANES, _compact, 0)

    cnt_pad = jnp.maximum(((cnt + 2 * CHUNK - 1) // (2 * CHUNK)) * (2 * CHUNK),
                          2 * CHUNK)
    nfill = (cnt_pad - cnt + LANES - 1) // LANES

    def _fill(k, _):
        dstv[pl.ds(cnt + k * LANES, LANES)] = jnp.full((LANES,), LTRASH,
                                                       jnp.int32)
        srcv[pl.ds(cnt + k * LANES, LANES)] = jnp.zeros((LANES,), jnp.int32)
        return 0

    lax.fori_loop(0, nfill, _fill, 0)
    nch = cnt_pad // CHUNK

    @pl.loop(0, CHUNK)
    def _zrow(i):
        @pl.loop(0, D, step=LANES)
        def _zcol(c):
            rb0[i, pl.ds(c, LANES)] = jnp.zeros((LANES,), jnp.float32)

    @pl.loop(0, ZPT // 8)
    def _zero(k):
        pltpu.sync_copy(rb0.at[pl.ds(0, 8)],
                        z_sh.at[pl.ds(sid * ZPT + k * 8, 8)])

    plsc.subcore_barrier()

    def _gather(j, rb, sem):
        return pltpu.async_copy(
            xs_hbm.at[srcv.at[pl.ds(j * CHUNK, CHUNK)]], rb, sem)

    def _gather_wait(j, rb, sem):
        pltpu.make_async_copy(
            xs_hbm.at[srcv.at[pl.ds(j * CHUNK, CHUNK)]], rb, sem).wait()

    def _scatter(j, rb, sem):
        return pltpu.async_copy(
            rb, z_sh.at[dstv.at[pl.ds(j * CHUNK, CHUNK)]], sem, add=True)

    def _scatter_wait(j, rb, sem):
        pltpu.make_async_copy(
            rb, z_sh.at[dstv.at[pl.ds(j * CHUNK, CHUNK)]], sem).wait()

    _gather(0, rb0, sem0)
    _gather(1, rb1, sem1)

    def _pipe(jj, _):
        j = jj * 2
        _gather_wait(j, rb0, sem0)
        _scatter(j, rb0, sem2)
        _gather_wait(j + 1, rb1, sem1)
        _scatter(j + 1, rb1, sem3)
        _scatter_wait(j, rb0, sem2)
        _gather(j + 2, rb0, sem0)
        _scatter_wait(j + 1, rb1, sem3)
        _gather(j + 3, rb1, sem1)
        return 0

    lax.fori_loop(0, (nch - 2) // 2, _pipe, 0)

    _gather_wait(nch - 2, rb0, sem0)
    _scatter(nch - 2, rb0, sem2)
    _gather_wait(nch - 1, rb1, sem1)
    _scatter(nch - 1, rb1, sem3)
    _scatter_wait(nch - 2, rb0, sem2)
    _scatter_wait(nch - 1, rb1, sem3)

    plsc.subcore_barrier()
    pltpu.sync_copy(z_sh.at[pl.ds(sid * WPT, WPT)],
                    z_hbm.at[pl.ds(base + sid * WPT, WPT)])


@jax.jit
def _agg_call(xs, src_slab, dst_slab):
    k = pl.kernel(
        _agg_body,
        out_type=jax.ShapeDtypeStruct((N_PAD, D), jnp.float32),
        mesh=_mesh(),
        compiler_params=_cp(),
        scratch_types=[
            pltpu.VMEM((EV + 2 * CHUNK,), jnp.int32),
            pltpu.VMEM((EV + 2 * CHUNK,), jnp.int32),
            pltpu.VMEM((CHUNK, D), jnp.float32),
            pltpu.VMEM((CHUNK, D), jnp.float32),
            pltpu.VMEM_SHARED((Z_ROWS, D), jnp.float32),
            pltpu.SemaphoreType.DMA,
            pltpu.SemaphoreType.DMA,
            pltpu.SemaphoreType.DMA,
            pltpu.SemaphoreType.DMA,
        ],
    )
    return k(xs, src_slab, dst_slab)


RB = 2000


def _scale_body(deg_ref, x_ref, xs_ref):
    d = deg_ref[0] + deg_ref[1] + 1.0
    xs_ref[...] = x_ref[...] * lax.rsqrt(d)


@jax.jit
def _scale_call(deg4, x):
    return pl.pallas_call(
        _scale_body,
        grid=(N // RB,),
        in_specs=[
            pl.BlockSpec((NC, RB, 1), lambda i: (0, i, 0)),
            pl.BlockSpec((RB, D), lambda i: (i, 0)),
        ],
        out_specs=pl.BlockSpec((RB, D), lambda i: (i, 0)),
        out_shape=jax.ShapeDtypeStruct((N, D), jnp.float32),
    )(deg4, x)


def _final_body(z_ref, xs_ref, deg_ref, w1_ref, b1_ref, w2_ref, b2_ref,
                o_ref):
    d = deg_ref[0] + deg_ref[1] + 1.0
    y = (z_ref[...] + xs_ref[...]) * lax.rsqrt(d)
    h1 = jnp.dot(y, w1_ref[...], preferred_element_type=jnp.float32,
                 precision="highest") + b1_ref[...]
    h2 = jnp.dot(y, w2_ref[...], preferred_element_type=jnp.float32,
                 precision="highest") + b2_ref[...]
    o_ref[...] = jnp.maximum(h1, 0.0) + h2


@jax.jit
def _final_call(z, xs, deg4, W1, b1, W2, b2):
    return pl.pallas_call(
        _final_body,
        grid=(N // RB,),
        in_specs=[
            pl.BlockSpec((RB, D), lambda i: (i, 0)),
            pl.BlockSpec((RB, D), lambda i: (i, 0)),
            pl.BlockSpec((NC, RB, 1), lambda i: (0, i, 0)),
            pl.BlockSpec((D, D), lambda i: (0, 0)),
            pl.BlockSpec((1, D), lambda i: (0, 0)),
            pl.BlockSpec((D, D), lambda i: (0, 0)),
            pl.BlockSpec((1, D), lambda i: (0, 0)),
        ],
        out_specs=pl.BlockSpec((RB, D), lambda i: (i, 0)),
        out_shape=jax.ShapeDtypeStruct((N, D), jnp.float32),
    )(z, xs, deg4, W1, b1, W2, b2)


def kernel(x, edge_index, W1, b1, W2, b2):
    ei = edge_index.astype(jnp.int32)
    pad2 = E_PAD2 - E
    src_slab = jnp.concatenate(
        [ei[0], jnp.zeros((pad2,), jnp.int32)]).reshape(NS, EV)
    dst_slab = jnp.concatenate(
        [ei[1], jnp.full((pad2,), TRASH, jnp.int32)]).reshape(NS, EV)

    deg4 = _deg_call(dst_slab).reshape(NC, HB * D, 1)
    xs = _scale_call(deg4, x)
    z = _agg_call(xs, src_slab, dst_slab)
    return _final_call(z, xs, deg4, W1, b1.reshape(1, D), W2,
                       b2.reshape(1, D))


# speedup vs baseline: 1.1006x; 1.1006x over previous
import dataclasses

import jax
import jax.numpy as jnp
from jax import lax
from jax.experimental import pallas as pl
from jax.experimental.pallas import tpu as pltpu
from jax.experimental.pallas import tpu_sc as plsc

N = 10000
D = 128
E = 320000

NC = 2
NS = 16
NW = NC * NS
CHUNK = 128
LANES = 16

NCHT = ((-(-E // (NS * CHUNK)) + 15) // 16) * 16
EV = NCHT * CHUNK
E_PAD2 = EV * NS

N_PAD = ((N + NS * CHUNK) // (NS * CHUNK)) * NS * CHUNK
RPT = N_PAD // NS
TRASH = 16383

HN = N_PAD // NC
Z_ROWS = HN + CHUNK
LTRASH = HN
ZPT = Z_ROWS // NS
WPT = HN // NS


def _mesh():
    return plsc.VectorSubcoreMesh(core_axis_name="c", subcore_axis_name="s",
                                  num_cores=NC, num_subcores=NS)


HB = 128


def _cp():
    cp = pltpu.CompilerParams()
    if "needs_layout_passes" in pltpu.CompilerParams.__dataclass_fields__:
        cp = dataclasses.replace(cp, needs_layout_passes=False)
    return cp


def _deg_body(dst_hbm, deg_hbm, dstv, idxrow, histo, deg_sh):
    cid = lax.axis_index("c")
    sid = lax.axis_index("s")
    pltpu.sync_copy(dst_hbm.at[sid, pl.ds(cid * (EV // 2), EV // 2)], dstv)

    @pl.loop(0, HB)
    def _zr(i):
        @pl.loop(0, D, step=LANES)
        def _zc(c):
            histo[i, pl.ds(c, LANES)] = jnp.zeros((LANES,), jnp.float32)

    @pl.loop(0, D, step=LANES)
    def _io(c):
        idxrow[0, pl.ds(c, LANES)] = lax.iota(jnp.int32, LANES) + c

    pltpu.sync_copy(histo.at[pl.ds(sid * (HB // NS), HB // NS)],
                    deg_sh.at[pl.ds(sid * (HB // NS), HB // NS)])
    plsc.subcore_barrier()

    @pl.loop(0, EV // 2, step=LANES)
    def _hist(c):
        v = dstv[pl.ds(c, LANES)]
        plsc.addupdate_scatter(histo, [v >> 7, v & 127],
                               jnp.ones((LANES,), jnp.float32))

    pltpu.sync_copy(histo, deg_sh.at[idxrow.at[0]], add=True)
    plsc.subcore_barrier()
    pltpu.sync_copy(deg_sh.at[pl.ds(sid * (HB // NS), HB // NS)],
                    deg_hbm.at[cid, pl.ds(sid * (HB // NS), HB // NS)])


@jax.jit
def _deg_call(dst_slab):
    k = pl.kernel(
        _deg_body,
        out_type=jax.ShapeDtypeStruct((NC, HB, D), jnp.float32),
        mesh=_mesh(),
        compiler_params=_cp(),
        scratch_types=[
            pltpu.VMEM((EV // 2,), jnp.int32),
            pltpu.VMEM((1, D), jnp.int32),
            pltpu.VMEM((HB, D), jnp.float32),
            pltpu.VMEM_SHARED((HB, D), jnp.float32),
        ],
    )
    return k(dst_slab)


def _agg_body(xs_hbm, src_hbm, dst_hbm, z_hbm, srcv, dstv, rb0, rb1, z_sh,
              sem0, sem1):
    cid = lax.axis_index("c")
    sid = lax.axis_index("s")
    base = cid * HN
    pltpu.sync_copy(src_hbm.at[sid], srcv.at[pl.ds(0, EV)])
    pltpu.sync_copy(dst_hbm.at[sid], dstv.at[pl.ds(0, EV)])

    def _compact(i, off):
        v = dstv[pl.ds(i * LANES, LANES)]
        s = srcv[pl.ds(i * LANES, LANES)]
        lv = v - base
        ok = (lv >= 0) & (lv < HN)
        plsc.store_compressed(dstv.at[pl.ds(off, LANES)], lv, mask=ok)
        plsc.store_compressed(srcv.at[pl.ds(off, LANES)], s, mask=ok)
        return off + jnp.sum(ok.astype(jnp.int32))

    cnt = lax.fori_loop(0, EV // LANES, _compact, 0)

    cnt_pad = jnp.maximum(((cnt + 2 * CHUNK - 1) // (2 * CHUNK)) * (2 * CHUNK),
                          2 * CHUNK)
    nfill = (cnt_pad - cnt + LANES - 1) // LANES

    def _fill(k, _):
        dstv[pl.ds(cnt + k * LANES, LANES)] = jnp.full((LANES,), LTRASH,
                                                       jnp.int32)
        srcv[pl.ds(cnt + k * LANES, LANES)] = jnp.zeros((LANES,), jnp.int32)
        return 0

    lax.fori_loop(0, nfill, _fill, 0)
    nch = cnt_pad // CHUNK

    @pl.loop(0, CHUNK)
    def _zrow(i):
        @pl.loop(0, D, step=LANES)
        def _zcol(c):
            rb0[i, pl.ds(c, LANES)] = jnp.zeros((LANES,), jnp.float32)

    @pl.loop(0, ZPT // CHUNK)
    def _zero(k):
        pltpu.sync_copy(rb0, z_sh.at[pl.ds(sid * ZPT + k * CHUNK, CHUNK)])

    pltpu.sync_copy(rb0.at[pl.ds(0, ZPT % CHUNK)],
                    z_sh.at[pl.ds(sid * ZPT + ZPT - ZPT % CHUNK,
                                  ZPT % CHUNK)])

    plsc.subcore_barrier()

    def _gather(j, rb, sem):
        return pltpu.async_copy(
            xs_hbm.at[srcv.at[pl.ds(j * CHUNK, CHUNK)]], rb, sem)

    def _gather_wait(j, rb, sem):
        pltpu.make_async_copy(
            xs_hbm.at[srcv.at[pl.ds(j * CHUNK, CHUNK)]], rb, sem).wait()

    def _scatter_sync(j, rb):
        pltpu.sync_copy(rb, z_sh.at[dstv.at[pl.ds(j * CHUNK, CHUNK)]],
                        add=True)

    _gather(0, rb0, sem0)

    def _pipe(jj, _):
        j = jj * 2
        _gather(j + 1, rb1, sem1)
        _gather_wait(j, rb0, sem0)
        _scatter_sync(j, rb0)
        _gather(j + 2, rb0, sem0)
        _gather_wait(j + 1, rb1, sem1)
        _scatter_sync(j + 1, rb1)
        return 0

    lax.fori_loop(0, (nch - 2) // 2, _pipe, 0)

    _gather(nch - 1, rb1, sem1)
    _gather_wait(nch - 2, rb0, sem0)
    _scatter_sync(nch - 2, rb0)
    _gather_wait(nch - 1, rb1, sem1)
    _scatter_sync(nch - 1, rb1)

    plsc.subcore_barrier()
    pltpu.sync_copy(z_sh.at[pl.ds(sid * WPT, WPT)],
                    z_hbm.at[pl.ds(base + sid * WPT, WPT)])


@jax.jit
def _agg_call(xs, src_slab, dst_slab):
    k = pl.kernel(
        _agg_body,
        out_type=jax.ShapeDtypeStruct((N_PAD, D), jnp.float32),
        mesh=_mesh(),
        compiler_params=_cp(),
        scratch_types=[
            pltpu.VMEM((EV + 2 * CHUNK,), jnp.int32),
            pltpu.VMEM((EV + 2 * CHUNK,), jnp.int32),
            pltpu.VMEM((CHUNK, D), jnp.float32),
            pltpu.VMEM((CHUNK, D), jnp.float32),
            pltpu.VMEM_SHARED((Z_ROWS, D), jnp.float32),
            pltpu.SemaphoreType.DMA,
            pltpu.SemaphoreType.DMA,
        ],
    )
    return k(xs, src_slab, dst_slab)


RB = 2000


def _scale_body(deg_ref, x_ref, xs_ref):
    d = deg_ref[0] + deg_ref[1] + 1.0
    xs_ref[...] = x_ref[...] * lax.rsqrt(d)


@jax.jit
def _scale_call(deg4, x):
    return pl.pallas_call(
        _scale_body,
        grid=(N // RB,),
        in_specs=[
            pl.BlockSpec((NC, RB, 1), lambda i: (0, i, 0)),
            pl.BlockSpec((RB, D), lambda i: (i, 0)),
        ],
        out_specs=pl.BlockSpec((RB, D), lambda i: (i, 0)),
        out_shape=jax.ShapeDtypeStruct((N, D), jnp.float32),
    )(deg4, x)


def _final_body(z_ref, xs_ref, deg_ref, w1_ref, b1_ref, w2_ref, b2_ref,
                o_ref):
    d = deg_ref[0] + deg_ref[1] + 1.0
    y = (z_ref[...] + xs_ref[...]) * lax.rsqrt(d)
    h1 = jnp.dot(y, w1_ref[...], preferred_element_type=jnp.float32,
                 precision="highest") + b1_ref[...]
    h2 = jnp.dot(y, w2_ref[...], preferred_element_type=jnp.float32,
                 precision="highest") + b2_ref[...]
    o_ref[...] = jnp.maximum(h1, 0.0) + h2


@jax.jit
def _final_call(z, xs, deg4, W1, b1, W2, b2):
    return pl.pallas_call(
        _final_body,
        grid=(N // RB,),
        in_specs=[
            pl.BlockSpec((RB, D), lambda i: (i, 0)),
            pl.BlockSpec((RB, D), lambda i: (i, 0)),
            pl.BlockSpec((NC, RB, 1), lambda i: (0, i, 0)),
            pl.BlockSpec((D, D), lambda i: (0, 0)),
            pl.BlockSpec((1, D), lambda i: (0, 0)),
            pl.BlockSpec((D, D), lambda i: (0, 0)),
            pl.BlockSpec((1, D), lambda i: (0, 0)),
        ],
        out_specs=pl.BlockSpec((RB, D), lambda i: (i, 0)),
        out_shape=jax.ShapeDtypeStruct((N, D), jnp.float32),
    )(z, xs, deg4, W1, b1, W2, b2)


def kernel(x, edge_index, W1, b1, W2, b2):
    ei = edge_index.astype(jnp.int32)
    pad2 = E_PAD2 - E
    src_slab = jnp.concatenate(
        [ei[0], jnp.zeros((pad2,), jnp.int32)]).reshape(NS, EV)
    dst_slab = jnp.concatenate(
        [ei[1], jnp.full((pad2,), TRASH, jnp.int32)]).reshape(NS, EV)

    deg4 = _deg_call(dst_slab).reshape(NC, HB * D, 1)
    xs = _scale_call(deg4, x)
    z = _agg_call(xs, src_slab, dst_slab)
    return _final_call(z, xs, deg4, W1, b1.reshape(1, D), W2,
                       b2.reshape(1, D))
